# Initial kernel scaffold; baseline (speedup 1.0000x reference)
import jax
import jax.numpy as jnp
from jax.experimental import pallas as pl


def kernel(edge_feat, edge_index, node_type):
    raise NotImplementedError("write your pallas kernel here")


# speedup vs baseline: 6.1635x; 6.1635x over previous
import functools

import jax
import jax.numpy as jnp
from jax import lax
from jax.experimental import pallas as pl
from jax.experimental.pallas import tpu as pltpu
from jax.experimental.pallas import tpu_sc as plsc

N_NODES = 10000
N_EDGES = 320000
D_FEAT = 128

N_CORES = 2
N_SUBCORES = 16
COLS = D_FEAT // N_CORES
EDGES_PER_TILE = N_EDGES // N_SUBCORES
BLOCK = 800
N_BLOCKS = EDGES_PER_TILE // BLOCK
SCATTER = 80
SC_PER_BLOCK = BLOCK // SCATTER
ROWS_PER_TILE = N_NODES // N_SUBCORES

_mesh = plsc.VectorSubcoreMesh(core_axis_name="c", subcore_axis_name="s")


@functools.partial(
    pl.kernel,
    out_type=jax.ShapeDtypeStruct((N_NODES, D_FEAT), jnp.float32),
    mesh=_mesh,
    scratch_types=[
        pltpu.VMEM((N_BLOCKS * SC_PER_BLOCK, SCATTER), jnp.int32),
        pltpu.VMEM((BLOCK, COLS), jnp.float32),
        pltpu.VMEM_SHARED((N_NODES, COLS), jnp.float32),
    ],
    compiler_params=pltpu.CompilerParams(use_tc_tiling_on_sc=False),
)
def _scatter_add(feat_hbm, idx_hbm, out_hbm, idx_v, feat_v, acc):
    c = lax.axis_index("c")
    s = lax.axis_index("s")
    col = c * COLS

    zv = jnp.zeros((16,), jnp.float32)

    def zero_body(i, carry):
        for k in range(COLS // 16):
            feat_v[i, pl.ds(k * 16, 16)] = zv
        return carry

    lax.fori_loop(0, ROWS_PER_TILE, zero_body, 0)
    pltpu.sync_copy(
        feat_v.at[pl.ds(0, ROWS_PER_TILE)],
        acc.at[pl.ds(s * ROWS_PER_TILE, ROWS_PER_TILE)],
    )
    plsc.subcore_barrier()

    pltpu.sync_copy(idx_hbm.at[s], idx_v)

    def body(j, carry):
        e0 = s * EDGES_PER_TILE + j * BLOCK
        pltpu.sync_copy(
            feat_hbm.at[pl.ds(e0, BLOCK), pl.ds(col, COLS)], feat_v
        )
        for k in range(SC_PER_BLOCK):
            pltpu.sync_copy(
                feat_v.at[pl.ds(k * SCATTER, SCATTER)],
                acc.at[idx_v.at[j * SC_PER_BLOCK + k]],
                add=True,
            )
        return carry

    lax.fori_loop(0, N_BLOCKS, body, 0)
    plsc.subcore_barrier()

    pltpu.sync_copy(
        acc.at[pl.ds(s * ROWS_PER_TILE, ROWS_PER_TILE)],
        out_hbm.at[pl.ds(s * ROWS_PER_TILE, ROWS_PER_TILE), pl.ds(col, COLS)],
    )


def kernel(edge_feat, edge_index, node_type):
    del node_type
    idx = edge_index[0].astype(jnp.int32).reshape(
        N_SUBCORES, N_BLOCKS * SC_PER_BLOCK, SCATTER
    )
    return _scatter_add(edge_feat, idx)
